# Initial kernel scaffold; baseline (speedup 1.0000x reference)
import jax
import jax.numpy as jnp
from jax.experimental import pallas as pl


def kernel(annotation, edge_index, etypes, W_et, b_et, w_ih, w_hh, b_ih, b_hh, gate_w, gate_b, out_w, out_b):
    raise NotImplementedError("write your pallas kernel here")


# speedup vs baseline: 10.5767x; 10.5767x over previous
import jax
import jax.numpy as jnp
from jax import lax
from jax.experimental import pallas as pl
from jax.experimental.pallas import tpu as pltpu
from jax.experimental.pallas import tpu_sc as plsc

N = 10000
E = 320000
ANN = 64
D = 128
T = 4
STEPS = 5
CLS = 10

NP = 10240
NH = NP // 2
NSUB = 16
ACC_R = 5184
ZT = ACC_R // NSUB
WT = NH // NSUB
CHUNK = 128
CT = 158
EW = CT * CHUNK
EP = NSUB * EW
ZR = 162
BR = 640


def _sc_body(trans_hbm, gidx_hbm, dst_hbm, out_hbm,
             gidx_v, dst_v, rows_v, zbuf, acc, sem):
    c = lax.axis_index("c")
    s = lax.axis_index("s")

    pltpu.sync_copy(gidx_hbm.at[s], gidx_v)
    pltpu.sync_copy(dst_hbm.at[c, s], dst_v)

    def zrow(r, carry):
        for k in range(D // 16):
            zbuf[r, pl.ds(k * 16, 16)] = jnp.zeros((16,), jnp.float32)
        return carry
    lax.fori_loop(0, ZR, zrow, 0)
    pltpu.sync_copy(zbuf, acc.at[pl.ds(s * ZT, ZR)])
    pltpu.sync_copy(zbuf, acc.at[pl.ds(s * ZT + ZR, ZR)])
    plsc.subcore_barrier()

    def chunk(j, carry):
        pltpu.async_copy(trans_hbm.at[gidx_v.at[j]], rows_v, sem).wait()
        pltpu.sync_copy(rows_v, acc.at[dst_v.at[j]], add=True)
        return carry
    lax.fori_loop(0, CT, chunk, 0)
    plsc.subcore_barrier()

    pltpu.sync_copy(acc.at[pl.ds(s * WT, WT)], out_hbm.at[c, pl.ds(s * WT, WT)])


_sc_scatter = pl.kernel(
    _sc_body,
    out_type=jax.ShapeDtypeStruct((2, NH, D), jnp.float32),
    mesh=plsc.VectorSubcoreMesh(core_axis_name="c", subcore_axis_name="s"),
    scratch_types=[
        pltpu.VMEM((CT, CHUNK), jnp.int32),
        pltpu.VMEM((CT, CHUNK), jnp.int32),
        pltpu.VMEM((CHUNK, D), jnp.float32),
        pltpu.VMEM((ZR, D), jnp.float32),
        pltpu.VMEM_SHARED((ACC_R, D), jnp.float32),
        pltpu.SemaphoreType.DMA,
    ],
)


def _split_trans(res, tr_ref):
    for t in range(T):
        tr_ref[t] = res[:, t * D:(t + 1) * D]


def _trans_body(h_ref, wcat_ref, bcat_ref, tr_ref):
    res = jnp.dot(h_ref[...], wcat_ref[...],
                  preferred_element_type=jnp.float32) + bcat_ref[...]
    _split_trans(res, tr_ref)


_k_trans = pl.pallas_call(
    _trans_body,
    grid=(NP // BR,),
    in_specs=[
        pl.BlockSpec((BR, D), lambda i: (i, 0)),
        pl.BlockSpec((D, T * D), lambda i: (0, 0)),
        pl.BlockSpec((1, T * D), lambda i: (0, 0)),
    ],
    out_specs=pl.BlockSpec((T, BR, D), lambda i: (0, i, 0)),
    out_shape=jax.ShapeDtypeStruct((T, NP, D), jnp.float32),
)


def _gru_core(a_ref, h_ref, wih_ref, whh_ref, bih_ref, bhh_ref):
    a = a_ref[...]
    h = h_ref[...]
    gi = jnp.dot(a, wih_ref[...], preferred_element_type=jnp.float32) + bih_ref[...]
    gh = jnp.dot(h, whh_ref[...], preferred_element_type=jnp.float32) + bhh_ref[...]
    r = jax.nn.sigmoid(gi[:, :D] + gh[:, :D])
    z = jax.nn.sigmoid(gi[:, D:2 * D] + gh[:, D:2 * D])
    n = jnp.tanh(gi[:, 2 * D:] + r * gh[:, 2 * D:])
    return (1.0 - z) * n + z * h


def _gru_trans_body(a_ref, h_ref, wih_ref, whh_ref, bih_ref, bhh_ref,
                    wcat_ref, bcat_ref, hn_ref, tr_ref):
    hn = _gru_core(a_ref, h_ref, wih_ref, whh_ref, bih_ref, bhh_ref)
    hn_ref[...] = hn
    res = jnp.dot(hn, wcat_ref[...],
                  preferred_element_type=jnp.float32) + bcat_ref[...]
    _split_trans(res, tr_ref)


_k_gru_trans = pl.pallas_call(
    _gru_trans_body,
    grid=(NP // BR,),
    in_specs=[
        pl.BlockSpec((BR, D), lambda i: (i, 0)),
        pl.BlockSpec((BR, D), lambda i: (i, 0)),
        pl.BlockSpec((D, 3 * D), lambda i: (0, 0)),
        pl.BlockSpec((D, 3 * D), lambda i: (0, 0)),
        pl.BlockSpec((1, 3 * D), lambda i: (0, 0)),
        pl.BlockSpec((1, 3 * D), lambda i: (0, 0)),
        pl.BlockSpec((D, T * D), lambda i: (0, 0)),
        pl.BlockSpec((1, T * D), lambda i: (0, 0)),
    ],
    out_specs=[
        pl.BlockSpec((BR, D), lambda i: (i, 0)),
        pl.BlockSpec((T, BR, D), lambda i: (0, i, 0)),
    ],
    out_shape=[
        jax.ShapeDtypeStruct((NP, D), jnp.float32),
        jax.ShapeDtypeStruct((T, NP, D), jnp.float32),
    ],
)


def _gru_body(a_ref, h_ref, wih_ref, whh_ref, bih_ref, bhh_ref, hn_ref):
    hn_ref[...] = _gru_core(a_ref, h_ref, wih_ref, whh_ref, bih_ref, bhh_ref)


_k_gru = pl.pallas_call(
    _gru_body,
    grid=(NP // BR,),
    in_specs=[
        pl.BlockSpec((BR, D), lambda i: (i, 0)),
        pl.BlockSpec((BR, D), lambda i: (i, 0)),
        pl.BlockSpec((D, 3 * D), lambda i: (0, 0)),
        pl.BlockSpec((D, 3 * D), lambda i: (0, 0)),
        pl.BlockSpec((1, 3 * D), lambda i: (0, 0)),
        pl.BlockSpec((1, 3 * D), lambda i: (0, 0)),
    ],
    out_specs=pl.BlockSpec((BR, D), lambda i: (i, 0)),
    out_shape=jax.ShapeDtypeStruct((NP, D), jnp.float32),
)


def _pool_body(h_ref, ann_ref, gwh_ref, gwa_ref, gb_ref, owh_ref, owa_ref,
               ob_ref, out_ref):
    h = h_ref[...]
    ann = ann_ref[...]
    g = (jnp.sum(h * gwh_ref[...], axis=1, keepdims=True)
         + jnp.sum(ann * gwa_ref[...], axis=1, keepdims=True) + gb_ref[0, 0])
    row = lax.broadcasted_iota(jnp.int32, (NP, 1), 0)
    g = jnp.where(row < N, g, -jnp.inf)
    m = jnp.max(g)
    w = jnp.exp(g - m)
    sw = jnp.sum(w)
    sh = jnp.sum(w * h, axis=0, keepdims=True)
    sa = jnp.sum(w * ann, axis=0, keepdims=True)
    logits = (jnp.dot(sh, owh_ref[...], preferred_element_type=jnp.float32)
              + jnp.dot(sa, owa_ref[...], preferred_element_type=jnp.float32))
    out_ref[...] = logits / sw + ob_ref[...]


_k_pool = pl.pallas_call(
    _pool_body,
    out_shape=jax.ShapeDtypeStruct((1, 128), jnp.float32),
)


@jax.jit
def _run(annotation, edge_index, etypes, W_et, b_et, w_ih, w_hh, b_ih, b_hh,
         gate_w, gate_b, out_w, out_b):
    h0 = jnp.zeros((NP, D), jnp.float32).at[:N, :ANN].set(annotation)
    ann_p = jnp.zeros((NP, ANN), jnp.float32).at[:N].set(annotation)
    src = edge_index[0].astype(jnp.int32)
    dst = edge_index[1].astype(jnp.int32)
    gidx = etypes.astype(jnp.int32) * NP + src
    gidx_p = jnp.zeros((EP,), jnp.int32).at[:E].set(gidx).reshape(NSUB, CT, CHUNK)
    dst_pad = jnp.full((EP,), NP, jnp.int32).at[:E].set(dst)
    locs = []
    for c in range(2):
        loc = dst_pad - c * NH
        loc = jnp.where((loc >= 0) & (loc < NH), loc, NH)
        locs.append(loc.reshape(NSUB, CT, CHUNK))
    dst_p = jnp.stack(locs)

    wcat = jnp.transpose(W_et, (2, 0, 1)).reshape(D, T * D)
    bcat = b_et.reshape(1, T * D)
    wih_t = w_ih.T
    whh_t = w_hh.T
    bih = b_ih.reshape(1, 3 * D)
    bhh = b_hh.reshape(1, 3 * D)
    gwh = gate_w[:, :D]
    gwa = gate_w[:, D:]
    gb = gate_b.reshape(1, 1)
    owh = jnp.zeros((D, 128), jnp.float32).at[:, :CLS].set(out_w[:, :D].T)
    owa = jnp.zeros((ANN, 128), jnp.float32).at[:, :CLS].set(out_w[:, D:].T)
    ob = jnp.zeros((1, 128), jnp.float32).at[0, :CLS].set(out_b)

    h = h0
    trans = _k_trans(h, wcat, bcat)
    for step in range(STEPS):
        halves = _sc_scatter(trans.reshape(T * NP, D), gidx_p, dst_p)
        a = halves.reshape(NP, D)
        if step < STEPS - 1:
            h, trans = _k_gru_trans(a, h, wih_t, whh_t, bih, bhh, wcat, bcat)
        else:
            h = _k_gru(a, h, wih_t, whh_t, bih, bhh)

    logits = _k_pool(h, ann_p, gwh, gwa, gb, owh, owa, ob)
    return logits[:, :CLS]


def kernel(annotation, edge_index, etypes, W_et, b_et, w_ih, w_hh, b_ih, b_hh,
           gate_w, gate_b, out_w, out_b):
    return _run(annotation, edge_index, etypes, W_et, b_et, w_ih, w_hh, b_ih,
                b_hh, gate_w, gate_b, out_w, out_b)
